# Initial kernel scaffold; baseline (speedup 1.0000x reference)
import jax
import jax.numpy as jnp
from jax.experimental import pallas as pl


def kernel(coords_2d, bias_table):
    raise NotImplementedError("write your pallas kernel here")


# speedup vs baseline: 23.1577x; 23.1577x over previous
import functools

import jax
import jax.numpy as jnp
from jax import lax
from jax.experimental import pallas as pl
from jax.experimental.pallas import tpu as pltpu
from jax.experimental.pallas import tpu_sc as plsc

NUM_HEADS = 16
NUM_BUCKETS = 32
MAX_DISTANCE = 128
B = 2
N = 1024

LUT_A = 256
LUT_SIZE = LUT_A * LUT_A
IDX_OFFSET = 127 * 256 + 127

LANES = 16
ROWS_PER_SLAB = 16
NUM_SLABS = N // ROWS_PER_SLAB
CHUNKS = N // LANES


def _lut_body(bias_ref, out_ref):
    nb = NUM_BUCKETS // 2
    max_exact = nb // 2

    def bucket(rel):
        n = -rel
        ret = (n < 0).astype(jnp.int32) * nb
        n = jnp.abs(n)
        is_small = n < max_exact
        safe_n = jnp.maximum(n, 1).astype(jnp.float32)
        val_if_large = max_exact + (
            jnp.log(safe_n / max_exact)
            / jnp.log(jnp.float32(MAX_DISTANCE / max_exact))
            * (nb - max_exact)
        ).astype(jnp.int32)
        val_if_large = jnp.minimum(val_if_large, nb - 1)
        return ret + jnp.where(is_small, n, val_if_large)

    f_col = bucket(lax.broadcasted_iota(jnp.int32, (LUT_A, 1), 0) - 127)
    f_row = bucket(lax.broadcasted_iota(jnp.int32, (1, LUT_A), 1) - 127)
    oh_a = (f_col == lax.broadcasted_iota(jnp.int32, (LUT_A, NUM_BUCKETS), 1))
    oh_a = oh_a.astype(jnp.float32)
    oh_bt = (lax.broadcasted_iota(jnp.int32, (NUM_BUCKETS, LUT_A), 0) == f_row)
    oh_bt = oh_bt.astype(jnp.float32)
    for h in range(NUM_HEADS):
        m = bias_ref[h]
        p = jnp.dot(oh_a, m, preferred_element_type=jnp.float32,
                    precision=lax.Precision.HIGHEST)
        out_ref[h] = jnp.dot(p, oh_bt, preferred_element_type=jnp.float32,
                             precision=lax.Precision.HIGHEST)


_build_lut = pl.pallas_call(
    _lut_body,
    out_shape=jax.ShapeDtypeStruct((NUM_HEADS, LUT_A, LUT_A), jnp.float32),
)


@functools.cache
def _make_sc_gather():
    mesh = plsc.VectorSubcoreMesh(core_axis_name="c", subcore_axis_name="s")
    return functools.partial(
        pl.kernel,
        mesh=mesh,
        out_type=jax.ShapeDtypeStruct((B * NUM_HEADS * N, N), jnp.float32),
        scratch_types=[
            pltpu.VMEM((2 * N,), jnp.float32),
            pltpu.VMEM((N,), jnp.int32),
            pltpu.VMEM((N,), jnp.int32),
            pltpu.VMEM((LUT_SIZE,), jnp.float32),
            pltpu.VMEM((ROWS_PER_SLAB, N), jnp.float32),
            pltpu.VMEM((ROWS_PER_SLAB, N), jnp.float32),
            pltpu.SemaphoreType.DMA,
            pltpu.SemaphoreType.DMA,
        ],
        compiler_params=pltpu.CompilerParams(needs_layout_passes=False),
    )(_sc_gather_body)


def _sc_gather_body(lut_hbm, coords_hbm, out_hbm,
               coords_v, p_v, q_v, lut_v, buf0, buf1, sem0, sem1):
    b = lax.axis_index("c")
    h = lax.axis_index("s")
    wid = b * NUM_HEADS + h

    pltpu.sync_copy(coords_hbm.at[b], coords_v)
    pltpu.sync_copy(lut_hbm.at[h], lut_v)

    lanes = lax.broadcasted_iota(jnp.int32, (LANES,), 0)

    def init_body(c, carry):
        base = c * LANES
        xi = plsc.load_gather(coords_v, [(lanes + base) * 2])
        yi = plsc.load_gather(coords_v, [(lanes + base) * 2 + 1])
        px = (xi * jnp.float32(MAX_DISTANCE)).astype(jnp.int32)
        py = (yi * jnp.float32(MAX_DISTANCE)).astype(jnp.int32)
        pv = px * 256 + py
        p_v[pl.ds(base, LANES)] = pv
        q_v[pl.ds(base, LANES)] = IDX_OFFSET - pv
        return carry

    lax.fori_loop(0, CHUNKS, init_body, 0)

    bufs = (buf0, buf1)
    sems = (sem0, sem1)
    out_row0 = wid * N

    def do_slab(s, buf):
        base_row = s * ROWS_PER_SLAB
        splats = [
            plsc.load_gather(p_v, [jnp.full((LANES,), base_row + r, jnp.int32)])
            for r in range(ROWS_PER_SLAB)
        ]

        def col_body(c, carry):
            q = q_v[pl.ds(c * LANES, LANES)]
            for r in range(ROWS_PER_SLAB):
                idx = splats[r] + q
                buf[r, pl.ds(c * LANES, LANES)] = plsc.load_gather(lut_v, [idx])
            return carry

        lax.fori_loop(0, CHUNKS, col_body, 0)

    def pair_body(ss, carry):
        for k in range(2):
            s = ss * 2 + k
            dst = out_hbm.at[pl.ds(out_row0 + s * ROWS_PER_SLAB, ROWS_PER_SLAB), :]

            @pl.when(ss > 0)
            def _wait():
                pltpu.make_async_copy(bufs[k], dst, sems[k]).wait()

            do_slab(s, bufs[k])
            pltpu.async_copy(bufs[k], dst, sems[k])
        return carry

    lax.fori_loop(0, NUM_SLABS // 2, pair_body, 0)

    tail = out_hbm.at[pl.ds(out_row0, ROWS_PER_SLAB), :]
    pltpu.make_async_copy(bufs[0], tail, sems[0]).wait()
    pltpu.make_async_copy(bufs[1], tail, sems[1]).wait()


def kernel(coords_2d, bias_table):
    bias_t = bias_table.T.reshape(NUM_HEADS, NUM_BUCKETS, NUM_BUCKETS)
    lut = _build_lut(bias_t).reshape(NUM_HEADS, LUT_SIZE)
    coords_flat = coords_2d.reshape(B, 2 * N)
    out = _make_sc_gather()(lut, coords_flat)
    return out.reshape(B, NUM_HEADS, N, N)


# speedup vs baseline: 79.3678x; 3.4273x over previous
import functools

import jax
import jax.numpy as jnp
from jax import lax
from jax.experimental import pallas as pl
from jax.experimental.pallas import tpu as pltpu
from jax.experimental.pallas import tpu_sc as plsc

NUM_HEADS = 16
NUM_BUCKETS = 32
MAX_DISTANCE = 128
B = 2
N = 1024

LUT_A = 256
LUT_SIZE = LUT_A * LUT_A
IDX_OFFSET = 127 * 256 + 127

LANES = 16
ROWS_PER_SLAB = 16
NUM_SLABS = N // ROWS_PER_SLAB
CHUNKS = N // LANES


def _lut_body(bias_ref, out_ref):
    nb = NUM_BUCKETS // 2
    max_exact = nb // 2

    def bucket(rel):
        n = -rel
        ret = (n < 0).astype(jnp.int32) * nb
        n = jnp.abs(n)
        is_small = n < max_exact
        safe_n = jnp.maximum(n, 1).astype(jnp.float32)
        val_if_large = max_exact + (
            jnp.log(safe_n / max_exact)
            / jnp.log(jnp.float32(MAX_DISTANCE / max_exact))
            * (nb - max_exact)
        ).astype(jnp.int32)
        val_if_large = jnp.minimum(val_if_large, nb - 1)
        return ret + jnp.where(is_small, n, val_if_large)

    f_col = bucket(lax.broadcasted_iota(jnp.int32, (LUT_A, 1), 0) - 127)
    f_row = bucket(lax.broadcasted_iota(jnp.int32, (1, LUT_A), 1) - 127)
    oh_a = (f_col == lax.broadcasted_iota(jnp.int32, (LUT_A, NUM_BUCKETS), 1))
    oh_a = oh_a.astype(jnp.float32)
    oh_bt = (lax.broadcasted_iota(jnp.int32, (NUM_BUCKETS, LUT_A), 0) == f_row)
    oh_bt = oh_bt.astype(jnp.float32)
    for h in range(NUM_HEADS):
        m = bias_ref[h]
        p = jnp.dot(oh_a, m, preferred_element_type=jnp.float32,
                    precision=lax.Precision.HIGHEST)
        out_ref[h] = jnp.dot(p, oh_bt, preferred_element_type=jnp.float32,
                             precision=lax.Precision.HIGHEST)


_build_lut = pl.pallas_call(
    _lut_body,
    out_shape=jax.ShapeDtypeStruct((NUM_HEADS, LUT_A, LUT_A), jnp.float32),
)


@functools.cache
def _make_sc_gather():
    mesh = plsc.VectorSubcoreMesh(core_axis_name="c", subcore_axis_name="s")
    return functools.partial(
        pl.kernel,
        mesh=mesh,
        out_type=jax.ShapeDtypeStruct((B * NUM_HEADS * N, N), jnp.float32),
        scratch_types=[
            pltpu.VMEM((2 * N,), jnp.float32),
            pltpu.VMEM((N,), jnp.int32),
            pltpu.VMEM((N,), jnp.int32),
            pltpu.VMEM((LUT_SIZE,), jnp.float32),
            pltpu.VMEM((ROWS_PER_SLAB, N), jnp.float32),
            pltpu.VMEM((ROWS_PER_SLAB, N), jnp.float32),
            pltpu.SemaphoreType.DMA,
            pltpu.SemaphoreType.DMA,
        ],
        compiler_params=pltpu.CompilerParams(needs_layout_passes=False),
    )(_sc_gather_body)


def _sc_gather_body(lut_hbm, coords_hbm, out_hbm,
               coords_v, p_v, q_v, lut_v, buf0, buf1, sem0, sem1):
    b = lax.axis_index("c")
    h = lax.axis_index("s")
    wid = b * NUM_HEADS + h

    pltpu.sync_copy(coords_hbm.at[b], coords_v)
    pltpu.sync_copy(lut_hbm.at[h], lut_v)

    lanes = lax.broadcasted_iota(jnp.int32, (LANES,), 0)

    @plsc.parallel_loop(0, CHUNKS, 1, unroll=2)
    def _init_body(c):
        base = c * LANES
        xi = plsc.load_gather(coords_v, [(lanes + base) * 2])
        yi = plsc.load_gather(coords_v, [(lanes + base) * 2 + 1])
        px = (xi * jnp.float32(MAX_DISTANCE)).astype(jnp.int32)
        py = (yi * jnp.float32(MAX_DISTANCE)).astype(jnp.int32)
        pv = px * 256 + py
        p_v[pl.ds(base, LANES)] = pv
        q_v[pl.ds(base, LANES)] = IDX_OFFSET - pv

    bufs = (buf0, buf1)
    sems = (sem0, sem1)
    out_row0 = wid * N

    def do_slab(s, buf):
        base_row = s * ROWS_PER_SLAB
        splats = [
            plsc.load_gather(p_v, [jnp.full((LANES,), base_row + r, jnp.int32)])
            for r in range(ROWS_PER_SLAB)
        ]

        @plsc.parallel_loop(0, CHUNKS, 1, unroll=2)
        def _col_body(c):
            q = q_v[pl.ds(c * LANES, LANES)]
            for r in range(ROWS_PER_SLAB):
                idx = splats[r] + q
                buf[r, pl.ds(c * LANES, LANES)] = plsc.load_gather(lut_v, [idx])

    def pair_body(ss, carry):
        for k in range(2):
            s = ss * 2 + k
            dst = out_hbm.at[pl.ds(out_row0 + s * ROWS_PER_SLAB, ROWS_PER_SLAB), :]

            @pl.when(ss > 0)
            def _wait():
                pltpu.make_async_copy(bufs[k], dst, sems[k]).wait()

            do_slab(s, bufs[k])
            pltpu.async_copy(bufs[k], dst, sems[k])
        return carry

    lax.fori_loop(0, NUM_SLABS // 2, pair_body, 0)

    tail = out_hbm.at[pl.ds(out_row0, ROWS_PER_SLAB), :]
    pltpu.make_async_copy(bufs[0], tail, sems[0]).wait()
    pltpu.make_async_copy(bufs[1], tail, sems[1]).wait()


def kernel(coords_2d, bias_table):
    bias_t = bias_table.T.reshape(NUM_HEADS, NUM_BUCKETS, NUM_BUCKETS)
    lut = _build_lut(bias_t).reshape(NUM_HEADS, LUT_SIZE)
    coords_flat = coords_2d.reshape(B, 2 * N)
    out = _make_sc_gather()(lut, coords_flat)
    return out.reshape(B, NUM_HEADS, N, N)
